# Initial kernel scaffold; baseline (speedup 1.0000x reference)
import jax
import jax.numpy as jnp
from jax.experimental import pallas as pl


def kernel(xyz, atom_xyz, atomtypes, batch, atom_batch, weights):
    raise NotImplementedError("write your pallas kernel here")


# speedup vs baseline: 1.0052x; 1.0052x over previous
import jax
import jax.numpy as jnp
from jax.experimental import pallas as pl


def _lk(x):
    return jax.nn.leaky_relu(x, negative_slope=0.2)


def _knn(x, y, k, chunk):
    N = x.shape[0]

    def f(xc):
        d2 = jnp.sum((xc[:, None, :] - y[None, :, :]) ** 2, axis=-1)
        _, idx = jax.lax.top_k(-d2, k)
        return idx
    idx = jax.lax.map(f, x.reshape(-1, chunk, x.shape[1]))
    idx = idx.reshape(N, k)
    x_ik = y[idx.reshape(-1)].reshape(N, k, y.shape[1])
    dists = jnp.sum((x[:, None, :] - x_ik) ** 2, axis=-1)
    return idx, dists


def _gn(x, gamma, beta, groups=2, eps=1e-5):
    n, c = x.shape
    xg = x.reshape(n, groups, c // groups)
    m = jnp.mean(xg, axis=-1, keepdims=True)
    v = jnp.var(xg, axis=-1, keepdims=True)
    xg = (xg - m) / jnp.sqrt(v + eps)
    return xg.reshape(n, c) * gamma + beta


def _mlp2(f, p):
    h = f @ p['W1'] + p['b1']
    h = _lk(h)
    return h @ p['W2'] + p['b2']


def _touch(x):
    return pl.pallas_call(
        lambda i, o: o.__setitem__((slice(None), slice(None)), i[:, :]),
        out_shape=jax.ShapeDtypeStruct(x.shape, x.dtype),
    )(x)


def kernel(xyz, atom_xyz, atomtypes, batch, atom_batch, weights):
    out = atomtypes @ weights['tt_W1'] + weights['tt_b1']
    out = _lk(out)
    out = out @ weights['tt_W2'] + weights['tt_b2']

    idx, dists = _knn(atom_xyz, atom_xyz, 17, 1024)
    idx = idx[:, 1:]
    dists = dists[:, 1:]
    k = 16
    m = out.shape[0]
    for i in range(3):
        p = weights['aa'][i]
        nd = out.shape[1]
        feats = out[idx.reshape(-1), :]
        feats = jnp.concatenate([feats, dists.reshape(-1, 1)], axis=1).reshape(m, k, nd + 1)
        feats = jnp.concatenate([jnp.repeat(out[:, None, :], k, axis=1), feats], axis=-1)
        msg = jnp.sum(_mlp2(feats, p), axis=1)
        out = out + _lk(_gn(msg, p['gamma'], p['beta']))

    idx2, dists2 = _knn(xyz, atom_xyz, 16, 1024)
    n = xyz.shape[0]
    nd = out.shape[1]
    pe = jnp.ones((n, nd), dtype=xyz.dtype)
    for i in range(3):
        p = weights['emb'][i]
        feats = out[idx2.reshape(-1), :]
        feats = jnp.concatenate([feats, dists2.reshape(-1, 1)], axis=1).reshape(n, 16, nd + 1)
        feats = jnp.concatenate([jnp.repeat(pe[:, None, :], 16, axis=1), feats], axis=-1)
        msg = jnp.sum(_mlp2(feats, p), axis=1)
        pe = pe + _lk(_gn(msg, p['gamma'], p['beta']))
    pe = _touch(pe)
    return pe


# speedup vs baseline: 5.5761x; 5.5473x over previous
import functools
import jax
import jax.numpy as jnp
from jax import lax
from jax.experimental import pallas as pl
from jax.experimental.pallas import tpu as pltpu
from jax.experimental.pallas import tpu_sc as plsc

F = 16
K = 16


def _leaky(x):
    return jnp.where(x >= 0, x, x * jnp.float32(0.2))


def _knn_body(k, x_ref, yt_ref, idx_ref, dist_ref):
    xq = x_ref[...]
    yt = yt_ref[...]
    q = xq.shape[0]
    m = yt.shape[1]
    d0 = xq[:, 0:1] - yt[0:1, :]
    d1 = xq[:, 1:2] - yt[1:2, :]
    d2 = xq[:, 2:3] - yt[2:3, :]
    d = (d0 * d0 + d1 * d1) + d2 * d2
    lanes = lax.broadcasted_iota(jnp.int32, (q, m), 1)
    for j in range(k):
        mn = jnp.min(d, axis=1, keepdims=True)
        cand = jnp.where(d == mn, lanes, m)
        ij = jnp.min(cand, axis=1, keepdims=True)
        idx_ref[:, j:j + 1] = ij
        dist_ref[:, j:j + 1] = mn
        d = jnp.where(lanes == ij, jnp.float32(3e38), d)


def _knn(x, y, k, bq):
    n = x.shape[0]
    m = y.shape[0]
    kern = pl.pallas_call(
        functools.partial(_knn_body, k),
        grid=(n // bq,),
        in_specs=[pl.BlockSpec((bq, 3), lambda i: (i, 0)),
                  pl.BlockSpec((3, m), lambda i: (0, 0))],
        out_specs=[pl.BlockSpec((bq, k), lambda i: (i, 0)),
                   pl.BlockSpec((bq, k), lambda i: (i, 0))],
        out_shape=[jax.ShapeDtypeStruct((n, k), jnp.int32),
                   jax.ShapeDtypeStruct((n, k), jnp.float32)],
    )
    return kern(x, y.T)


def _sc_gather(table, idx):
    e = idx.shape[0]
    info = plsc.get_sparse_core_info()
    nc = info.num_cores
    nw = nc * info.num_subcores
    bpw = e // nw
    chunk = min(bpw, 512)
    nchunk = bpw // chunk
    mesh = plsc.VectorSubcoreMesh(core_axis_name="c", subcore_axis_name="s")

    @functools.partial(
        pl.kernel, mesh=mesh,
        out_type=jax.ShapeDtypeStruct((e, 128), jnp.float32),
        scratch_types=[pltpu.VMEM((chunk,), jnp.int32),
                       pltpu.VMEM((chunk, 128), jnp.float32),
                       pltpu.SemaphoreType.DMA],
    )
    def gk(table_hbm, idx_hbm, out_hbm, idx_v, rows_v, sem):
        wid = lax.axis_index("s") * nc + lax.axis_index("c")
        base = wid * bpw
        for c in range(nchunk):
            off = base + c * chunk
            pltpu.sync_copy(idx_hbm.at[pl.ds(off, chunk)], idx_v)
            pltpu.async_copy(table_hbm.at[idx_v], rows_v, sem).wait()
            pltpu.sync_copy(rows_v, out_hbm.at[pl.ds(off, chunk)])

    return gk(table, idx)


def _mp_body(self_ref, g_ref, dist_ref, wa_ref, wb_ref, w2_ref, cv_ref, out_ref):
    s = self_ref[...]
    wa = wa_ref[...]
    wb = wb_ref[...]
    w2 = w2_ref[...]
    b1 = cv_ref[0:1, :]
    b2 = cv_ref[1:2, :]
    gamma = cv_ref[2:3, :]
    beta = cv_ref[3:4, :]
    q = s.shape[0]
    sv = jnp.dot(s, wa, preferred_element_type=jnp.float32) + b1
    lanes = lax.broadcasted_iota(jnp.int32, (q, F), 1)
    acc = jnp.zeros((q, F), jnp.float32)
    for kk in range(K):
        gk = g_ref[kk][:, 0:F]
        ef = jnp.where(lanes == 6, dist_ref[:, kk:kk + 1], gk)
        tk = jnp.dot(ef, wb, preferred_element_type=jnp.float32)
        acc = acc + _leaky(sv + tk)
    msg = jnp.dot(acc, w2, preferred_element_type=jnp.float32) + jnp.float32(K) * b2
    m1 = ((msg[:, 0:1] + msg[:, 1:2]) + msg[:, 2:3]) / 3.0
    m2 = ((msg[:, 3:4] + msg[:, 4:5] + msg[:, 5:6])) / 3.0
    e0 = msg[:, 0:1] - m1
    e1 = msg[:, 1:2] - m1
    e2 = msg[:, 2:3] - m1
    f0 = msg[:, 3:4] - m2
    f1 = msg[:, 4:5] - m2
    f2 = msg[:, 5:6] - m2
    v1 = ((e0 * e0 + e1 * e1) + e2 * e2) / 3.0
    v2 = ((f0 * f0 + f1 * f1) + f2 * f2) / 3.0
    mfull = jnp.where(lanes < 3, m1, jnp.where(lanes < 6, m2, jnp.float32(0.0)))
    vfull = jnp.where(lanes < 3, v1, jnp.where(lanes < 6, v2, jnp.float32(1.0)))
    xn = (msg - mfull) / jnp.sqrt(vfull + jnp.float32(1e-5)) * gamma + beta
    out_ref[...] = s + _leaky(xn)


def _mp_layer(feat, g3, dist, wa, wb, w2, cv, bq):
    n = feat.shape[0]
    kern = pl.pallas_call(
        _mp_body,
        grid=(n // bq,),
        in_specs=[pl.BlockSpec((bq, F), lambda i: (i, 0)),
                  pl.BlockSpec((K, bq, 128), lambda i: (0, i, 0)),
                  pl.BlockSpec((bq, K), lambda i: (i, 0)),
                  pl.BlockSpec((F, F), lambda i: (0, 0)),
                  pl.BlockSpec((F, F), lambda i: (0, 0)),
                  pl.BlockSpec((F, F), lambda i: (0, 0)),
                  pl.BlockSpec((8, F), lambda i: (0, 0))],
        out_specs=pl.BlockSpec((bq, F), lambda i: (i, 0)),
        out_shape=jax.ShapeDtypeStruct((n, F), jnp.float32),
    )
    return kern(feat, g3, dist, wa, wb, w2, cv)


def _tt_body(a_ref, w1_ref, w2_ref, cv_ref, out_ref):
    a = a_ref[...]
    b1 = cv_ref[0:1, :]
    b2 = cv_ref[1:2, :]
    h = _leaky(jnp.dot(a, w1_ref[...], preferred_element_type=jnp.float32) + b1)
    out_ref[...] = jnp.dot(h, w2_ref[...], preferred_element_type=jnp.float32) + b2


def _tt_mlp(a, w1, w2, cv, bq):
    n = a.shape[0]
    kern = pl.pallas_call(
        _tt_body,
        grid=(n // bq,),
        in_specs=[pl.BlockSpec((bq, F), lambda i: (i, 0)),
                  pl.BlockSpec((F, F), lambda i: (0, 0)),
                  pl.BlockSpec((F, F), lambda i: (0, 0)),
                  pl.BlockSpec((8, F), lambda i: (0, 0))],
        out_specs=pl.BlockSpec((bq, F), lambda i: (i, 0)),
        out_shape=jax.ShapeDtypeStruct((n, F), jnp.float32),
    )
    return kern(a, w1, w2, cv)


def _pad2(a, r, c):
    return jnp.zeros((r, c), jnp.float32).at[:a.shape[0], :a.shape[1]].set(a)


def _layer_consts(p, d):
    wa = _pad2(p['W1'][:d, :], F, F)
    wb = (jnp.zeros((F, F), jnp.float32)
          .at[:d, :2 * d + 1].set(p['W1'][d:2 * d, :])
          .at[d, :2 * d + 1].set(p['W1'][2 * d, :]))
    w2 = _pad2(p['W2'], F, F)
    cv = (jnp.zeros((8, F), jnp.float32)
          .at[0, :2 * d + 1].set(p['b1'])
          .at[1, :d].set(p['b2'])
          .at[2, :d].set(p['gamma'])
          .at[3, :d].set(p['beta']))
    return wa, wb, w2, cv


def kernel(xyz, atom_xyz, atomtypes, batch, atom_batch, weights):
    w = weights
    natoms = atom_xyz.shape[0]
    npts = xyz.shape[0]
    d = atomtypes.shape[1]

    at_p = jnp.pad(atomtypes, ((0, 0), (0, F - d)))
    ttcv = (jnp.zeros((8, F), jnp.float32)
            .at[0, :d].set(w['tt_b1']).at[1, :d].set(w['tt_b2']))
    out = _tt_mlp(at_p, _pad2(w['tt_W1'], F, F), _pad2(w['tt_W2'], F, F),
                  ttcv, bq=512)

    idx_a, dist_a = _knn(atom_xyz, atom_xyz, K + 1, 256)
    idx_a = idx_a[:, 1:]
    dist_a = dist_a[:, 1:]
    idx_a_flat = idx_a.T.reshape(-1)

    for i in range(3):
        wa, wb, w2, cv = _layer_consts(w['aa'][i], d)
        tab = jnp.pad(out, ((0, 0), (0, 128 - F)))
        g = _sc_gather(tab, idx_a_flat).reshape(K, natoms, 128)
        out = _mp_layer(out, g, dist_a, wa, wb, w2, cv, bq=512)

    idx_p, dist_p = _knn(xyz, atom_xyz, K, 256)
    idx_p_flat = idx_p.T.reshape(-1)
    tab = jnp.pad(out, ((0, 0), (0, 128 - F)))
    g2 = _sc_gather(tab, idx_p_flat).reshape(K, npts, 128)

    pe = jnp.pad(jnp.ones((npts, d), jnp.float32), ((0, 0), (0, F - d)))
    for i in range(3):
        wa, wb, w2, cv = _layer_consts(w['emb'][i], d)
        pe = _mp_layer(pe, g2, dist_p, wa, wb, w2, cv, bq=512)
    return pe[:, :d]
